# baseline (speedup 1.0000x reference)
import functools

import jax
import jax.numpy as jnp
from jax import lax
from jax.experimental import pallas as pl
from jax.experimental.pallas import tpu as pltpu
from jax.experimental.pallas import tpu_sc as plsc

_MASK_ID = 103
_NC, _NS = 2, 16
_NW = _NC * _NS


def kernel(inputs_embeds, word_emb, prompt_emb):
    B, S, H = inputs_embeds.shape
    P = prompt_emb.shape[0]
    T = 1 + P + S
    dt = inputs_embeds.dtype
    RB = S // _NW
    CH = RB // 2
    RP = P // 8
    mal, mof = divmod(_MASK_ID, 8)
    mal *= 8

    mesh = plsc.VectorSubcoreMesh(core_axis_name="c", subcore_axis_name="s")

    @functools.partial(
        pl.kernel,
        out_type=jax.ShapeDtypeStruct((T, B, H), dt),
        mesh=mesh,
        scratch_types=[
            pltpu.VMEM((CH, H), dt),
            pltpu.VMEM((CH, H), dt),
            pltpu.VMEM((RP, H), dt),
            pltpu.VMEM((8, H), dt),
            pltpu.VMEM((1, H), dt),
            pltpu.SemaphoreType.DMA,
            pltpu.SemaphoreType.DMA,
            pltpu.SemaphoreType.DMA,
        ],
    )
    def body(in_hbm, word_hbm, prompt_hbm, out_hbm,
             va, vb, vp, vw, vw1, g_sem, s_sem, p_sem):
        wid = lax.axis_index("s") * _NC + lax.axis_index("c")

        @pl.when(wid < 8)
        def _prompt():
            p0 = pl.multiple_of(wid * RP, 8)
            pltpu.async_copy(prompt_hbm.at[pl.ds(p0, RP)], vp, p_sem).wait()
            for b in range(B):
                pltpu.async_copy(
                    vp, out_hbm.at[pl.ds(1 + p0, RP), b], p_sem
                ).wait()

        @pl.when(wid == 8)
        def _mask():
            pltpu.async_copy(word_hbm.at[pl.ds(mal, 8)], vw, p_sem).wait()
            for c in range(0, H, 16):
                vw1[0, pl.ds(c, 16)] = vw[mof, pl.ds(c, 16)]
            for b in range(B):
                pltpu.async_copy(vw1, out_hbm.at[pl.ds(0, 1), b], p_sem).wait()

        s0 = pl.multiple_of(wid * RB, 8)
        t0 = 1 + P + wid * RB
        bufs = (va, vb)
        chunks = [(b, h) for b in range(B) for h in range(RB // CH)]
        prev_s = None
        cur_g = pltpu.async_copy(
            in_hbm.at[chunks[0][0], pl.ds(s0 + chunks[0][1] * CH, CH)],
            bufs[0], g_sem,
        )
        for i, (b, h) in enumerate(chunks):
            cur_g.wait()
            if prev_s is not None:
                prev_s.wait()
            if i + 1 < len(chunks):
                nb, nh = chunks[i + 1]
                cur_g = pltpu.async_copy(
                    in_hbm.at[nb, pl.ds(s0 + nh * CH, CH)],
                    bufs[(i + 1) % 2], g_sem,
                )
            prev_s = pltpu.async_copy(
                bufs[i % 2], out_hbm.at[pl.ds(t0 + h * CH, CH), b], s_sem
            )
        prev_s.wait()

    res = body(inputs_embeds, word_emb, prompt_emb)
    return jnp.transpose(res, (1, 0, 2))


# speedup vs baseline: 1.0106x; 1.0106x over previous
import functools

import jax
import jax.numpy as jnp
from jax import lax
from jax.experimental import pallas as pl
from jax.experimental.pallas import tpu as pltpu
from jax.experimental.pallas import tpu_sc as plsc

_MASK_ID = 103
_NC, _NS = 2, 16
_NW = _NC * _NS


def kernel(inputs_embeds, word_emb, prompt_emb):
    B, S, H = inputs_embeds.shape
    P = prompt_emb.shape[0]
    T = 1 + P + S
    dt = inputs_embeds.dtype
    RB = S // _NW
    CH = RB // 2
    RP = P // 8
    mal, mof = divmod(_MASK_ID, 8)
    mal *= 8

    mesh = plsc.VectorSubcoreMesh(core_axis_name="c", subcore_axis_name="s")

    @functools.partial(
        pl.kernel,
        out_type=jax.ShapeDtypeStruct((T, B, H), dt),
        mesh=mesh,
        scratch_types=[
            pltpu.VMEM((CH, H), dt),
            pltpu.VMEM((CH, H), dt),
            pltpu.VMEM((RP, H), dt),
            pltpu.VMEM((8, H), dt),
            pltpu.VMEM((1, H), dt),
            pltpu.SemaphoreType.DMA,
            pltpu.SemaphoreType.DMA,
            pltpu.SemaphoreType.DMA,
        ],
    )
    def body(in_hbm, word_hbm, prompt_hbm, out_hbm,
             va, vb, vp, vw, vw1, g_sem, s_sem, p_sem):
        wid = lax.axis_index("s") * _NC + lax.axis_index("c")

        @pl.when(wid < 8)
        def _prompt():
            p0 = pl.multiple_of(wid * RP, 8)
            pltpu.async_copy(prompt_hbm.at[pl.ds(p0, RP)], vp, p_sem).wait()
            for b in range(B):
                pltpu.async_copy(vp, out_hbm.at[pl.ds(1 + p0, RP), b], p_sem)

        @pl.when(wid == 8)
        def _mask():
            pltpu.async_copy(word_hbm.at[pl.ds(mal, 8)], vw, p_sem).wait()
            for c in range(0, H, 16):
                vw1[0, pl.ds(c, 16)] = vw[mof, pl.ds(c, 16)]
            for b in range(B):
                pltpu.async_copy(vw1, out_hbm.at[pl.ds(0, 1), b], p_sem)

        s0 = pl.multiple_of(wid * RB, 8)
        t0 = 1 + P + wid * RB
        bufs = (va, vb)
        chunks = [(b, h) for b in range(B) for h in range(RB // CH)]
        prev_s = None
        cur_g = pltpu.async_copy(
            in_hbm.at[chunks[0][0], pl.ds(s0 + chunks[0][1] * CH, CH)],
            bufs[0], g_sem,
        )
        for i, (b, h) in enumerate(chunks):
            cur_g.wait()
            if prev_s is not None:
                prev_s.wait()
            if i + 1 < len(chunks):
                nb, nh = chunks[i + 1]
                cur_g = pltpu.async_copy(
                    in_hbm.at[nb, pl.ds(s0 + nh * CH, CH)],
                    bufs[(i + 1) % 2], g_sem,
                )
            prev_s = pltpu.async_copy(
                bufs[i % 2], out_hbm.at[pl.ds(t0 + h * CH, CH), b], s_sem
            )
        prev_s.wait()

        @pl.when(wid < 8)
        def _drain_prompt():
            p0 = pl.multiple_of(wid * RP, 8)
            for b in range(B):
                pltpu.make_async_copy(
                    vp, out_hbm.at[pl.ds(1 + p0, RP), b], p_sem
                ).wait()

        @pl.when(wid == 8)
        def _drain_mask():
            for b in range(B):
                pltpu.make_async_copy(
                    vw1, out_hbm.at[pl.ds(0, 1), b], p_sem
                ).wait()

    res = body(inputs_embeds, word_emb, prompt_emb)
    return jnp.transpose(res, (1, 0, 2))
